# baseline (speedup 1.0000x reference)
import functools

import jax
import jax.numpy as jnp
from jax.experimental import pallas as pl
from jax.experimental.pallas import tpu as pltpu

KNN_K = 16


def _pairwise_sqdist(a, b):
    aa = jnp.sum(a * a, axis=-1)[:, :, None]
    bb = jnp.sum(b * b, axis=-1)[:, None, :]
    return aa + bb - 2.0 * jnp.einsum('bnd,bmd->bnm', a, b)


def _knn(q, ref, k):
    d = _pairwise_sqdist(q, ref)
    neg, idx = jax.lax.top_k(-d, k)
    return idx, jnp.maximum(-neg, 0.0)


def _gather(x, idx):
    return jax.vmap(lambda xb, ib: xb[ib])(x, idx)


def _pt_block(xyz, feat, p, n_sample):
    q = feat @ p['Wq']
    kf = feat @ p['Wk']
    vf = feat @ p['Wv']
    idx, _ = _knn(xyz, xyz, n_sample)
    kg = _gather(kf, idx)
    vg = _gather(vf, idx)
    pos = xyz[:, :, None, :] - _gather(xyz, idx)
    pe = jax.nn.relu(pos @ p['P1'] + p['pb1']) @ p['P2'] + p['pb2']
    a = jax.nn.relu((q[:, :, None, :] - kg + pe) @ p['A1'] + p['ab1']) @ p['A2'] + p['ab2']
    a = jax.nn.softmax(a, axis=2)
    out = jnp.sum(a * (vg + pe), axis=2)
    return xyz, out @ p['Wo'] + p['ob'] + feat


def _tr_block(xyz, feat, p, n_sample, fps_rate):
    new_xyz = xyz[:, ::fps_rate, :]
    idx, _ = _knn(new_xyz, xyz, n_sample)
    f = jax.nn.relu(feat @ p['W'] + p['b'])
    new_feat = jnp.max(_gather(f, idx), axis=2)
    return _pt_block(new_xyz, new_feat, p['pt'], n_sample)


def _tu_layer(cx, fx, cf, ff, p, k=3):
    idx, d = _knn(fx, cx, k)
    w = 1.0 / (d + 1e-3)
    w = w / jnp.sum(w, axis=-1, keepdims=True)
    interp = jnp.sum(_gather(cf, idx) * w[..., None], axis=2)
    return fx, interp @ p['W1'] + p['b1'] + ff @ p['W2'] + p['b2']


def _head_kernel(f_ref, w1_ref, b1_ref, w2_ref, b2_ref, o_ref):
    h = f_ref[0] @ w1_ref[...] + b1_ref[...]
    o_ref[0] = h @ w2_ref[...] + b2_ref[...]


def _head(f8, p):
    B, N, C = f8.shape
    NC = p['fc2_W'].shape[1]
    out = pl.pallas_call(
        _head_kernel,
        out_shape=jax.ShapeDtypeStruct((B, N, NC), jnp.float32),
        grid=(B,),
        in_specs=[
            pl.BlockSpec((1, N, C), lambda b: (b, 0, 0)),
            pl.BlockSpec((C, C), lambda b: (0, 0)),
            pl.BlockSpec((C,), lambda b: (0,)),
            pl.BlockSpec((C, NC), lambda b: (0, 0)),
            pl.BlockSpec((NC,), lambda b: (0,)),
        ],
        out_specs=pl.BlockSpec((1, N, NC), lambda b: (b, 0, 0)),
    )(f8, p['fc1_W'], p['fc1_b'], p['fc2_W'], p['fc2_b'])
    return out


def kernel(inputs, params):
    p = params
    x = jnp.transpose(inputs, (0, 2, 1))
    xyz = x[..., :3]
    feat = (x @ p['in_W'] + p['in_b']) * p['bn_g'] + p['bn_b']
    xyz0, f0 = _pt_block(xyz, feat, p['pt0'], KNN_K)
    xyz1, f1 = _tr_block(xyz0, f0, p['tr1'], KNN_K, 2)
    xyz2, f2 = _tr_block(xyz1, f1, p['tr2'], KNN_K, 2)
    xyz3, f3 = _tr_block(xyz2, f2, p['tr3'], KNN_K, 2)
    xyz4, f4 = _tr_block(xyz3, f3, p['tr4'], KNN_K, 2)
    f4 = f4 @ p['mid_W'] + p['mid_b']
    xyz4, f4 = _pt_block(xyz4, f4, p['ptm'], KNN_K)
    x5, f5 = _tu_layer(xyz4, xyz3, f4, f3, p['tu5'])
    x5, f5 = _pt_block(x5, f5, p['pt5'], KNN_K)
    x6, f6 = _tu_layer(x5, xyz2, f5, f2, p['tu6'])
    x6, f6 = _pt_block(x6, f6, p['pt6'], KNN_K)
    x7, f7 = _tu_layer(x6, xyz1, f6, f1, p['tu7'])
    x7, f7 = _pt_block(x7, f7, p['pt7'], KNN_K)
    x8, f8 = _tu_layer(x7, xyz0, f7, f0, p['tu8'])
    x8, f8 = _pt_block(x8, f8, p['pt8'], KNN_K)
    return _head(f8, p)


# speedup vs baseline: 2.3706x; 2.3706x over previous
import functools

import jax
import jax.numpy as jnp
from jax.experimental import pallas as pl
from jax.experimental.pallas import tpu as pltpu

KNN_K = 16


def _knn_body(q_ref, r_ref, idx_ref, d_ref, *, k, nr, qb):
    q = q_ref[0]
    r = r_ref[0]
    aa = jnp.sum(q * q, axis=1, keepdims=True)
    bb = jnp.sum(r * r, axis=1)[None, :]
    ab = jax.lax.dot_general(q, r, (((1,), (1,)), ((), ())),
                             precision=jax.lax.Precision.DEFAULT)
    d = aa + bb - 2.0 * ab
    col = jax.lax.broadcasted_iota(jnp.int32, (qb, nr), 1)
    idx_cols = []
    d_cols = []
    for _ in range(k):
        m = jnp.min(d, axis=1, keepdims=True)
        sel = jnp.min(jnp.where(d == m, col, nr), axis=1, keepdims=True)
        idx_cols.append(sel)
        d_cols.append(jnp.maximum(m, 0.0))
        d = jnp.where(col == sel, jnp.float32(jnp.inf), d)
    idx_ref[0] = jnp.concatenate(idx_cols, axis=1)
    d_ref[0] = jnp.concatenate(d_cols, axis=1)


def _knn(q, ref, k):
    B, Nq, _ = q.shape
    Nr = ref.shape[1]
    QB = min(Nq, 512)
    kern = functools.partial(_knn_body, k=k, nr=Nr, qb=QB)
    idx, dist = pl.pallas_call(
        kern,
        out_shape=(jax.ShapeDtypeStruct((B, Nq, k), jnp.int32),
                   jax.ShapeDtypeStruct((B, Nq, k), jnp.float32)),
        grid=(B, Nq // QB),
        in_specs=[
            pl.BlockSpec((1, QB, 3), lambda b, i: (b, i, 0)),
            pl.BlockSpec((1, Nr, 3), lambda b, i: (b, 0, 0)),
        ],
        out_specs=(pl.BlockSpec((1, QB, k), lambda b, i: (b, i, 0)),
                   pl.BlockSpec((1, QB, k), lambda b, i: (b, i, 0))),
    )(q, ref)
    return idx, dist


def _gather(x, idx):
    return jax.vmap(lambda xb, ib: xb[ib])(x, idx)


_DOT = functools.partial(jnp.dot, precision=jax.lax.Precision.DEFAULT)


def _attn_body(xyz_ref, feat_ref, q_ref, g_ref,
               p1_ref, pb1_ref, p2_ref, pb2_ref,
               a1_ref, ab1_ref, a2_ref, ab2_ref,
               wo_ref, ob_ref, out_ref, *, qb, c, k):
    g = g_ref[0]
    kg = g[:, :c]
    vg = g[:, c:2 * c]
    xg = g[:, 2 * c:2 * c + 3]
    xyz = xyz_ref[0]
    xyzr = jnp.broadcast_to(xyz[:, None, :], (qb, k, 3)).reshape(qb * k, 3)
    pos = xyzr - xg
    pe = _DOT(jax.nn.relu(_DOT(pos, p1_ref[...]) + pb1_ref[...]),
              p2_ref[...]) + pb2_ref[...]
    q = q_ref[0]
    qr = jnp.broadcast_to(q[:, None, :], (qb, k, c)).reshape(qb * k, c)
    t = _DOT(jax.nn.relu(_DOT(qr - kg + pe, a1_ref[...]) + ab1_ref[...]),
             a2_ref[...]) + ab2_ref[...]
    t3 = t.reshape(qb, k, c)
    m = jnp.max(t3, axis=1, keepdims=True)
    e = jnp.exp(t3 - m)
    s = jnp.sum(e, axis=1, keepdims=True)
    a = e / s
    vpe3 = (vg + pe).reshape(qb, k, c)
    o = jnp.sum(a * vpe3, axis=1)
    out_ref[0] = _DOT(o, wo_ref[...]) + ob_ref[...] + feat_ref[0]


def _attn(xyz, feat, q, g, p, k):
    B, N, C = feat.shape
    D = g.shape[-1]
    QB = min(N, max(64, 32768 // C))
    kern = functools.partial(_attn_body, qb=QB, c=C, k=k)
    wspec = lambda sh: pl.BlockSpec(sh, lambda b, i: tuple(0 for _ in sh))
    out = pl.pallas_call(
        kern,
        out_shape=jax.ShapeDtypeStruct((B, N, C), jnp.float32),
        grid=(B, N // QB),
        in_specs=[
            pl.BlockSpec((1, QB, 3), lambda b, i: (b, i, 0)),
            pl.BlockSpec((1, QB, C), lambda b, i: (b, i, 0)),
            pl.BlockSpec((1, QB, C), lambda b, i: (b, i, 0)),
            pl.BlockSpec((1, QB * k, D), lambda b, i: (b, i, 0)),
            wspec((3, C)), wspec((C,)), wspec((C, C)), wspec((C,)),
            wspec((C, C)), wspec((C,)), wspec((C, C)), wspec((C,)),
            wspec((C, C)), wspec((C,)),
        ],
        out_specs=pl.BlockSpec((1, QB, C), lambda b, i: (b, i, 0)),
    )(xyz, feat, q, g, p['P1'], p['pb1'], p['P2'], p['pb2'],
      p['A1'], p['ab1'], p['A2'], p['ab2'], p['Wo'], p['ob'])
    return out


def _gather_table(table, idx):
    B, Nq, K = idx.shape
    g = _gather(table, idx.reshape(B, Nq * K))
    return g


def _pt_block(xyz, feat, p, n_sample):
    B, N, C = feat.shape
    q = feat @ p['Wq']
    kf = feat @ p['Wk']
    vf = feat @ p['Wv']
    idx, _ = _knn(xyz, xyz, n_sample)
    xyz_pad = jnp.pad(xyz, ((0, 0), (0, 0), (0, 13)))
    table = jnp.concatenate([kf, vf, xyz_pad], axis=-1)
    g = _gather_table(table, idx)
    out = _attn(xyz, feat, q, g, p, n_sample)
    return xyz, out


def _tr_block(xyz, feat, p, n_sample, fps_rate):
    new_xyz = xyz[:, ::fps_rate, :]
    idx, _ = _knn(new_xyz, xyz, n_sample)
    f = jax.nn.relu(feat @ p['W'] + p['b'])
    new_feat = jnp.max(_gather(f, idx), axis=2)
    return _pt_block(new_xyz, new_feat, p['pt'], n_sample)


def _tu_layer(cx, fx, cf, ff, p, k=3):
    idx, d = _knn(fx, cx, k)
    w = 1.0 / (d + 1e-3)
    w = w / jnp.sum(w, axis=-1, keepdims=True)
    interp = jnp.sum(_gather(cf, idx) * w[..., None], axis=2)
    return fx, interp @ p['W1'] + p['b1'] + ff @ p['W2'] + p['b2']


def _head_kernel(f_ref, w1_ref, b1_ref, w2_ref, b2_ref, o_ref):
    h = f_ref[0] @ w1_ref[...] + b1_ref[...]
    o_ref[0] = h @ w2_ref[...] + b2_ref[...]


def _head(f8, p):
    B, N, C = f8.shape
    NC = p['fc2_W'].shape[1]
    out = pl.pallas_call(
        _head_kernel,
        out_shape=jax.ShapeDtypeStruct((B, N, NC), jnp.float32),
        grid=(B,),
        in_specs=[
            pl.BlockSpec((1, N, C), lambda b: (b, 0, 0)),
            pl.BlockSpec((C, C), lambda b: (0, 0)),
            pl.BlockSpec((C,), lambda b: (0,)),
            pl.BlockSpec((C, NC), lambda b: (0, 0)),
            pl.BlockSpec((NC,), lambda b: (0,)),
        ],
        out_specs=pl.BlockSpec((1, N, NC), lambda b: (b, 0, 0)),
    )(f8, p['fc1_W'], p['fc1_b'], p['fc2_W'], p['fc2_b'])
    return out


def kernel(inputs, params):
    p = params
    x = jnp.transpose(inputs, (0, 2, 1))
    xyz = x[..., :3]
    feat = (x @ p['in_W'] + p['in_b']) * p['bn_g'] + p['bn_b']
    xyz0, f0 = _pt_block(xyz, feat, p['pt0'], KNN_K)
    xyz1, f1 = _tr_block(xyz0, f0, p['tr1'], KNN_K, 2)
    xyz2, f2 = _tr_block(xyz1, f1, p['tr2'], KNN_K, 2)
    xyz3, f3 = _tr_block(xyz2, f2, p['tr3'], KNN_K, 2)
    xyz4, f4 = _tr_block(xyz3, f3, p['tr4'], KNN_K, 2)
    f4 = f4 @ p['mid_W'] + p['mid_b']
    xyz4, f4 = _pt_block(xyz4, f4, p['ptm'], KNN_K)
    x5, f5 = _tu_layer(xyz4, xyz3, f4, f3, p['tu5'])
    x5, f5 = _pt_block(x5, f5, p['pt5'], KNN_K)
    x6, f6 = _tu_layer(x5, xyz2, f5, f2, p['tu6'])
    x6, f6 = _pt_block(x6, f6, p['pt6'], KNN_K)
    x7, f7 = _tu_layer(x6, xyz1, f6, f1, p['tu7'])
    x7, f7 = _pt_block(x7, f7, p['pt7'], KNN_K)
    x8, f8 = _tu_layer(x7, xyz0, f7, f0, p['tu8'])
    x8, f8 = _pt_block(x8, f8, p['pt8'], KNN_K)
    return _head(f8, p)


# speedup vs baseline: 7.1285x; 3.0070x over previous
import functools

import jax
import jax.numpy as jnp
from jax.experimental import pallas as pl
from jax.experimental.pallas import tpu as pltpu
from jax.experimental.pallas import tpu_sc as plsc

KNN_K = 16


def _knn_body(q_ref, r_ref, idx_ref, d_ref, *, k, nr, qb):
    q = q_ref[0]
    r = r_ref[0]
    aa = jnp.sum(q * q, axis=1, keepdims=True)
    bb = jnp.sum(r * r, axis=1)[None, :]
    ab = jax.lax.dot_general(q, r, (((1,), (1,)), ((), ())),
                             precision=jax.lax.Precision.DEFAULT)
    d = aa + bb - 2.0 * ab
    col = jax.lax.broadcasted_iota(jnp.int32, (qb, nr), 1)
    idx_cols = []
    d_cols = []
    for _ in range(k):
        m = jnp.min(d, axis=1, keepdims=True)
        sel = jnp.min(jnp.where(d == m, col, nr), axis=1, keepdims=True)
        idx_cols.append(sel)
        d_cols.append(jnp.maximum(m, 0.0))
        d = jnp.where(col == sel, jnp.float32(jnp.inf), d)
    idx_ref[0] = jnp.concatenate(idx_cols, axis=1)
    d_ref[0] = jnp.concatenate(d_cols, axis=1)


def _knn(q, ref, k):
    B, Nq, _ = q.shape
    Nr = ref.shape[1]
    QB = min(Nq, 512)
    kern = functools.partial(_knn_body, k=k, nr=Nr, qb=QB)
    idx, dist = pl.pallas_call(
        kern,
        out_shape=(jax.ShapeDtypeStruct((B, Nq, k), jnp.int32),
                   jax.ShapeDtypeStruct((B, Nq, k), jnp.float32)),
        grid=(B, Nq // QB),
        in_specs=[
            pl.BlockSpec((1, QB, 3), lambda b, i: (b, i, 0)),
            pl.BlockSpec((1, Nr, 3), lambda b, i: (b, 0, 0)),
        ],
        out_specs=(pl.BlockSpec((1, QB, k), lambda b, i: (b, i, 0)),
                   pl.BlockSpec((1, QB, k), lambda b, i: (b, i, 0))),
    )(q, ref)
    return idx, dist


def _gather(x, idx):
    return jax.vmap(lambda xb, ib: xb[ib])(x, idx)


_DOT = functools.partial(jnp.dot, precision=jax.lax.Precision.DEFAULT)


def _attn_body(xyz_ref, feat_ref, q_ref, g_ref,
               p1_ref, pb1_ref, p2_ref, pb2_ref,
               a1_ref, ab1_ref, a2_ref, ab2_ref,
               wo_ref, ob_ref, out_ref, *, qb, c, k):
    g = g_ref[0]
    kg = g[:, :c]
    vg = g[:, c:2 * c]
    xg = g[:, 2 * c:2 * c + 3]
    xyz = xyz_ref[0]
    xyzr = jnp.broadcast_to(xyz[:, None, :], (qb, k, 3)).reshape(qb * k, 3)
    pos = xyzr - xg
    pe = _DOT(jax.nn.relu(_DOT(pos, p1_ref[...]) + pb1_ref[...]),
              p2_ref[...]) + pb2_ref[...]
    q = q_ref[0]
    qr = jnp.broadcast_to(q[:, None, :], (qb, k, c)).reshape(qb * k, c)
    t = _DOT(jax.nn.relu(_DOT(qr - kg + pe, a1_ref[...]) + ab1_ref[...]),
             a2_ref[...]) + ab2_ref[...]
    t3 = t.reshape(qb, k, c)
    m = jnp.max(t3, axis=1, keepdims=True)
    e = jnp.exp(t3 - m)
    s = jnp.sum(e, axis=1, keepdims=True)
    a = e / s
    vpe3 = (vg + pe).reshape(qb, k, c)
    o = jnp.sum(a * vpe3, axis=1)
    out_ref[0] = _DOT(o, wo_ref[...]) + ob_ref[...] + feat_ref[0]


def _attn(xyz, feat, q, g, p, k):
    B, N, C = feat.shape
    D = g.shape[-1]
    QB = min(N, max(64, 32768 // C))
    kern = functools.partial(_attn_body, qb=QB, c=C, k=k)
    wspec = lambda sh: pl.BlockSpec(sh, lambda b, i: tuple(0 for _ in sh))
    out = pl.pallas_call(
        kern,
        out_shape=jax.ShapeDtypeStruct((B, N, C), jnp.float32),
        grid=(B, N // QB),
        in_specs=[
            pl.BlockSpec((1, QB, 3), lambda b, i: (b, i, 0)),
            pl.BlockSpec((1, QB, C), lambda b, i: (b, i, 0)),
            pl.BlockSpec((1, QB, C), lambda b, i: (b, i, 0)),
            pl.BlockSpec((1, QB * k, D), lambda b, i: (b, i, 0)),
            wspec((3, C)), wspec((C,)), wspec((C, C)), wspec((C,)),
            wspec((C, C)), wspec((C,)), wspec((C, C)), wspec((C,)),
            wspec((C, C)), wspec((C,)),
        ],
        out_specs=pl.BlockSpec((1, QB, C), lambda b, i: (b, i, 0)),
    )(xyz, feat, q, g, p['P1'], p['pb1'], p['P2'], p['pb2'],
      p['A1'], p['ab1'], p['A2'], p['ab2'], p['Wo'], p['ob'])
    return out


_SC_NW = 32


def _sc_gather_flat(table, idx):
    R_out = idx.shape[0]
    D = table.shape[1]
    b_per_w = R_out // _SC_NW
    chunk = min(b_per_w, max(8, 1 << ((49152 // D).bit_length() - 1)))
    n_chunks = b_per_w // chunk
    mesh = plsc.VectorSubcoreMesh(core_axis_name="c", subcore_axis_name="s")

    @functools.partial(
        pl.kernel, mesh=mesh,
        out_type=jax.ShapeDtypeStruct((R_out, D), jnp.float32),
        scratch_types=[
            pltpu.VMEM((chunk,), jnp.int32),
            pltpu.VMEM((chunk, D), jnp.float32),
            pltpu.SemaphoreType.DMA,
        ],
    )
    def gk(tab_hbm, idx_hbm, out_hbm, idx_v, rows_v, sem):
        wid = jax.lax.axis_index("s") * 2 + jax.lax.axis_index("c")
        base = wid * b_per_w
        for ci in range(n_chunks):
            off = base + ci * chunk
            pltpu.sync_copy(idx_hbm.at[pl.ds(off, chunk)], idx_v)
            pltpu.async_copy(tab_hbm.at[idx_v], rows_v, sem).wait()
            pltpu.sync_copy(rows_v, out_hbm.at[pl.ds(off, chunk)])

    return gk(table, idx)


def _gather_table(table, idx):
    B, Nr, D = table.shape
    _, Nq, K = idx.shape
    idxg = (idx + (jnp.arange(B, dtype=jnp.int32) * Nr)[:, None, None])
    out = _sc_gather_flat(table.reshape(B * Nr, D),
                          idxg.reshape(B * Nq * K))
    return out.reshape(B, Nq * K, D)


def _pt_block(xyz, feat, p, n_sample):
    B, N, C = feat.shape
    q = feat @ p['Wq']
    kf = feat @ p['Wk']
    vf = feat @ p['Wv']
    idx, _ = _knn(xyz, xyz, n_sample)
    Dp = -(-(2 * C + 3) // 128) * 128
    xyz_pad = jnp.pad(xyz, ((0, 0), (0, 0), (0, Dp - 2 * C - 3)))
    table = jnp.concatenate([kf, vf, xyz_pad], axis=-1)
    g = _gather_table(table, idx)
    out = _attn(xyz, feat, q, g, p, n_sample)
    return xyz, out


def _tr_block(xyz, feat, p, n_sample, fps_rate):
    new_xyz = xyz[:, ::fps_rate, :]
    idx, _ = _knn(new_xyz, xyz, n_sample)
    f = jax.nn.relu(feat @ p['W'] + p['b'])
    new_feat = jnp.max(_gather(f, idx), axis=2)
    return _pt_block(new_xyz, new_feat, p['pt'], n_sample)


def _tu_layer(cx, fx, cf, ff, p, k=3):
    idx, d = _knn(fx, cx, k)
    w = 1.0 / (d + 1e-3)
    w = w / jnp.sum(w, axis=-1, keepdims=True)
    interp = jnp.sum(_gather(cf, idx) * w[..., None], axis=2)
    return fx, interp @ p['W1'] + p['b1'] + ff @ p['W2'] + p['b2']


def _head_kernel(f_ref, w1_ref, b1_ref, w2_ref, b2_ref, o_ref):
    h = f_ref[0] @ w1_ref[...] + b1_ref[...]
    o_ref[0] = h @ w2_ref[...] + b2_ref[...]


def _head(f8, p):
    B, N, C = f8.shape
    NC = p['fc2_W'].shape[1]
    out = pl.pallas_call(
        _head_kernel,
        out_shape=jax.ShapeDtypeStruct((B, N, NC), jnp.float32),
        grid=(B,),
        in_specs=[
            pl.BlockSpec((1, N, C), lambda b: (b, 0, 0)),
            pl.BlockSpec((C, C), lambda b: (0, 0)),
            pl.BlockSpec((C,), lambda b: (0,)),
            pl.BlockSpec((C, NC), lambda b: (0, 0)),
            pl.BlockSpec((NC,), lambda b: (0,)),
        ],
        out_specs=pl.BlockSpec((1, N, NC), lambda b: (b, 0, 0)),
    )(f8, p['fc1_W'], p['fc1_b'], p['fc2_W'], p['fc2_b'])
    return out


def kernel(inputs, params):
    p = params
    x = jnp.transpose(inputs, (0, 2, 1))
    xyz = x[..., :3]
    feat = (x @ p['in_W'] + p['in_b']) * p['bn_g'] + p['bn_b']
    xyz0, f0 = _pt_block(xyz, feat, p['pt0'], KNN_K)
    xyz1, f1 = _tr_block(xyz0, f0, p['tr1'], KNN_K, 2)
    xyz2, f2 = _tr_block(xyz1, f1, p['tr2'], KNN_K, 2)
    xyz3, f3 = _tr_block(xyz2, f2, p['tr3'], KNN_K, 2)
    xyz4, f4 = _tr_block(xyz3, f3, p['tr4'], KNN_K, 2)
    f4 = f4 @ p['mid_W'] + p['mid_b']
    xyz4, f4 = _pt_block(xyz4, f4, p['ptm'], KNN_K)
    x5, f5 = _tu_layer(xyz4, xyz3, f4, f3, p['tu5'])
    x5, f5 = _pt_block(x5, f5, p['pt5'], KNN_K)
    x6, f6 = _tu_layer(x5, xyz2, f5, f2, p['tu6'])
    x6, f6 = _pt_block(x6, f6, p['pt6'], KNN_K)
    x7, f7 = _tu_layer(x6, xyz1, f6, f1, p['tu7'])
    x7, f7 = _pt_block(x7, f7, p['pt7'], KNN_K)
    x8, f8 = _tu_layer(x7, xyz0, f7, f0, p['tu8'])
    x8, f8 = _pt_block(x8, f8, p['pt8'], KNN_K)
    return _head(f8, p)
